# baseline (speedup 1.0000x reference)
import functools

import jax
import jax.numpy as jnp
from jax import lax
from jax.experimental import pallas as pl
from jax.experimental.pallas import tpu as pltpu
from jax.experimental.pallas import tpu_sc as plsc

N = 10000
E = 320000
D = 128

NC = 2
NS = 16
NW = NC * NS

CHUNK = 128
NCHUNK = 80
EW = CHUNK * NCHUNK
E_PAD = NW * EW
N_ACC = 10240
JUNK_ROW = N
RW = N_ACC // NS

NBUF = 2


def _segsum_kernel(x_hbm, src_hbm, dst_hbm, out_hbm,
                   ibuf, dst_v, rows_v, acc_sh, *sems):
    gsem = sems[0:2]
    ssem = sems[2:4]
    isem = sems[4:8]
    cid = lax.axis_index("c")
    sid = lax.axis_index("s")
    wid = sid * NC + cid

    zvec = jnp.zeros((16,), jnp.float32)

    def zbody(r, carry):
        for j in range(D // 16):
            rows_v[0, r, pl.ds(j * 16, 16)] = zvec
        return carry

    lax.fori_loop(0, CHUNK, zbody, 0)
    for z in range(RW // CHUNK):
        pltpu.sync_copy(rows_v.at[0],
                        acc_sh.at[pl.ds(sid * RW + z * CHUNK, CHUNK)])
    plsc.subcore_barrier()

    pltpu.sync_copy(dst_hbm.at[wid], dst_v)

    def fire_idx(c, j):
        pltpu.async_copy(src_hbm.at[wid, c], ibuf.at[j], isem[j])

    def wait_idx(c, j):
        pltpu.make_async_copy(src_hbm.at[wid, c], ibuf.at[j],
                              isem[j]).wait()

    def fire_gather(b, k):
        pltpu.async_copy(x_hbm.at[ibuf.at[k]], rows_v.at[b], gsem[b])

    def wait_gather(b):
        pltpu.make_async_copy(x_hbm.at[ibuf.at[0]], rows_v.at[b],
                              gsem[b]).wait()

    def fire_scatter(c, b):
        pltpu.async_copy(rows_v.at[b], acc_sh.at[dst_v.at[c]], ssem[b],
                         add=True)

    def wait_scatter(c, b):
        pltpu.make_async_copy(rows_v.at[b], acc_sh.at[dst_v.at[c]],
                              ssem[b]).wait()

    def step(c, k, fi=True):
        b = k % 2
        wait_scatter(c - 2, b)
        wait_idx(c, k)
        fire_gather(b, k)
        if fi:
            fire_idx(c + 2, (k + 2) % 4)
        wait_gather(1 - b)
        fire_scatter(c - 1, 1 - b)

    fire_idx(0, 0)
    fire_idx(1, 1)
    wait_idx(0, 0)
    fire_gather(0, 0)
    fire_idx(2, 2)
    wait_idx(1, 1)
    fire_gather(1, 1)
    fire_idx(3, 3)
    wait_gather(0)
    fire_scatter(0, 0)
    step(2, 2)
    step(3, 3)

    def body(g, carry):
        for k in range(4):
            step(g * 4 + k, k)
        return carry

    lax.fori_loop(1, NCHUNK // 4 - 1, body, 0)

    step(NCHUNK - 4, 0)
    step(NCHUNK - 3, 1)
    step(NCHUNK - 2, 2, fi=False)
    step(NCHUNK - 1, 3, fi=False)
    wait_gather(1)
    fire_scatter(NCHUNK - 1, 1)
    wait_scatter(NCHUNK - 2, 0)
    wait_scatter(NCHUNK - 1, 1)

    plsc.subcore_barrier()

    pltpu.sync_copy(acc_sh.at[pl.ds(sid * RW, RW)],
                    out_hbm.at[cid, pl.ds(sid * RW, RW)])


_segsum = functools.partial(
    pl.kernel,
    out_type=jax.ShapeDtypeStruct((NC, N_ACC, D), jnp.float32),
    mesh=plsc.VectorSubcoreMesh(core_axis_name="c", subcore_axis_name="s"),
    scratch_types=(
        [
            pltpu.VMEM((4, CHUNK), jnp.int32),
            pltpu.VMEM((NCHUNK, CHUNK), jnp.int32),
            pltpu.VMEM((NBUF, CHUNK, D), jnp.float32),
            pltpu.VMEM_SHARED((N_ACC, D), jnp.float32),
        ]
        + [pltpu.SemaphoreType.DMA] * 8
    ),
)(_segsum_kernel)


BM = 512


def _gc_body(p_ref, w1_ref, b1_ref, w2_ref, o_ref):
    s = p_ref[0] + p_ref[1]
    h = jnp.dot(s, w1_ref[...], preferred_element_type=jnp.float32,
                precision=jax.lax.Precision.HIGHEST) + b1_ref[...]
    h = jnp.maximum(h, 0.0)
    o_ref[...] = jnp.dot(h, w2_ref[...], preferred_element_type=jnp.float32,
                         precision=jax.lax.Precision.HIGHEST)


def _fin_body(p_ref, b2_ref, w3_ref, b3_ref, o_ref):
    h = jnp.maximum(p_ref[0] + p_ref[1] + b2_ref[...], 0.0)
    o_ref[...] = jnp.sum(h * w3_ref[...], axis=1, keepdims=True) + b3_ref[...]


def kernel(x, adj, W1, b1, W2, b2, W3, b3):
    src = adj[0]
    dst = adj[1]
    pad = E_PAD - E
    src_p = jnp.concatenate([src, jnp.zeros((pad,), jnp.int32)])
    dst_p = jnp.concatenate([dst, jnp.full((pad,), JUNK_ROW, jnp.int32)])
    src_p = src_p.reshape(NW, NCHUNK, CHUNK)
    dst_p = dst_p.reshape(NW, NCHUNK, CHUNK)

    parts1 = _segsum(x, src_p, dst_p)

    support2 = pl.pallas_call(
        _gc_body,
        grid=(pl.cdiv(N, BM),),
        in_specs=[
            pl.BlockSpec((NC, BM, D), lambda i: (0, i, 0)),
            pl.BlockSpec((D, 2 * D), lambda i: (0, 0)),
            pl.BlockSpec((1, 2 * D), lambda i: (0, 0)),
            pl.BlockSpec((2 * D, D), lambda i: (0, 0)),
        ],
        out_specs=pl.BlockSpec((BM, D), lambda i: (i, 0)),
        out_shape=jax.ShapeDtypeStruct((N, D), jnp.float32),
    )(parts1, W1, b1.reshape(1, -1), W2)

    parts2 = _segsum(support2, src_p, dst_p)

    out = pl.pallas_call(
        _fin_body,
        grid=(pl.cdiv(N, BM),),
        in_specs=[
            pl.BlockSpec((NC, BM, D), lambda i: (0, i, 0)),
            pl.BlockSpec((1, D), lambda i: (0, 0)),
            pl.BlockSpec((1, D), lambda i: (0, 0)),
            pl.BlockSpec((1, 1), lambda i: (0, 0)),
        ],
        out_specs=pl.BlockSpec((BM, 1), lambda i: (i, 0)),
        out_shape=jax.ShapeDtypeStruct((N, 1), jnp.float32),
    )(parts2, b2.reshape(1, -1), W3.T, b3.reshape(1, 1))

    return out


# speedup vs baseline: 1.1800x; 1.1800x over previous
import functools

import jax
import jax.numpy as jnp
from jax import lax
from jax.experimental import pallas as pl
from jax.experimental.pallas import tpu as pltpu
from jax.experimental.pallas import tpu_sc as plsc

N = 10000
E = 320000
D = 128

NC = 2
NS = 16
NW = NC * NS

CHUNK = 80
NCHUNK = 128
EW = CHUNK * NCHUNK
E_PAD = NW * EW
N_ACC = 10240
JUNK_ROW = N
RW = N_ACC // NS

NBUF = 4
NIB = 8
LAG = 2


def _segsum_kernel(x_hbm, src_hbm, dst_hbm, out_hbm,
                   ibuf_s, ibuf_d, rows_v, acc_sh, *sems):
    gsem = sems[0:NBUF]
    ssem = sems[NBUF:2 * NBUF]
    isem = sems[2 * NBUF:2 * NBUF + NIB]
    cid = lax.axis_index("c")
    sid = lax.axis_index("s")
    wid = sid * NC + cid

    zvec = jnp.zeros((16,), jnp.float32)

    def zbody(r, carry):
        for j in range(D // 16):
            rows_v[0, r, pl.ds(j * 16, 16)] = zvec
        return carry

    lax.fori_loop(0, CHUNK, zbody, 0)
    for z in range(RW // CHUNK):
        pltpu.sync_copy(rows_v.at[0],
                        acc_sh.at[pl.ds(sid * RW + z * CHUNK, CHUNK)])
    plsc.subcore_barrier()

    def fire_idx(c, j):
        pltpu.async_copy(src_hbm.at[wid, c], ibuf_s.at[j], isem[j])
        pltpu.async_copy(dst_hbm.at[wid, c], ibuf_d.at[j], isem[j])

    def wait_idx(c, j):
        pltpu.make_async_copy(src_hbm.at[wid, c], ibuf_s.at[j],
                              isem[j]).wait()
        pltpu.make_async_copy(dst_hbm.at[wid, c], ibuf_d.at[j],
                              isem[j]).wait()

    def fire_gather(b, k):
        pltpu.async_copy(x_hbm.at[ibuf_s.at[k]], rows_v.at[b], gsem[b])

    def wait_gather(b):
        pltpu.make_async_copy(x_hbm.at[ibuf_s.at[0]], rows_v.at[b],
                              gsem[b]).wait()

    def fire_scatter(jb, b):
        pltpu.async_copy(rows_v.at[b], acc_sh.at[ibuf_d.at[jb]], ssem[b],
                         add=True)

    def wait_scatter(b):
        pltpu.make_async_copy(rows_v.at[b], acc_sh.at[ibuf_d.at[0]],
                              ssem[b]).wait()

    def emit(c, m, first=False, fi=True, fg=True):
        k = m % NBUF
        j = m % NIB
        if fg:
            if not first:
                wait_scatter(k)
            wait_idx(c, j)
            fire_gather(k, j)
            if fi:
                fire_idx(c + NBUF, (j + NBUF) % NIB)
        cs = c - LAG
        if not isinstance(cs, int) or cs >= 0:
            kb = (m - LAG) % NBUF
            wait_gather(kb)
            fire_scatter((m - LAG) % NIB, kb)

    for j in range(NBUF):
        fire_idx(j, j)
    for c in range(NIB):
        emit(c, c, first=(c < NBUF))

    def body(g, carry):
        for m in range(NIB):
            emit(g * NIB + m, m)
        return carry

    lax.fori_loop(1, NCHUNK // NIB - 1, body, 0)

    for c in range(NCHUNK - NIB, NCHUNK):
        emit(c, c % NIB, fi=(c + NBUF < NCHUNK))
    for c in range(NCHUNK, NCHUNK + LAG):
        emit(c, c % NIB, fg=False)
    for cs in range(NCHUNK - NBUF, NCHUNK):
        wait_scatter(cs % NBUF)

    plsc.subcore_barrier()

    pltpu.sync_copy(acc_sh.at[pl.ds(sid * RW, RW)],
                    out_hbm.at[cid, pl.ds(sid * RW, RW)])


_segsum = functools.partial(
    pl.kernel,
    out_type=jax.ShapeDtypeStruct((NC, N_ACC, D), jnp.float32),
    mesh=plsc.VectorSubcoreMesh(core_axis_name="c", subcore_axis_name="s"),
    scratch_types=(
        [
            pltpu.VMEM((NIB, CHUNK), jnp.int32),
            pltpu.VMEM((NIB, CHUNK), jnp.int32),
            pltpu.VMEM((NBUF, CHUNK, D), jnp.float32),
            pltpu.VMEM_SHARED((N_ACC, D), jnp.float32),
        ]
        + [pltpu.SemaphoreType.DMA] * (2 * NBUF + NIB)
    ),
)(_segsum_kernel)


BM = 512


def _gc_body(p_ref, w1_ref, b1_ref, w2_ref, o_ref):
    s = p_ref[0] + p_ref[1]
    h = jnp.dot(s, w1_ref[...], preferred_element_type=jnp.float32,
                precision=jax.lax.Precision.HIGHEST) + b1_ref[...]
    h = jnp.maximum(h, 0.0)
    o_ref[...] = jnp.dot(h, w2_ref[...], preferred_element_type=jnp.float32,
                         precision=jax.lax.Precision.HIGHEST)


def _fin_body(p_ref, b2_ref, w3_ref, b3_ref, o_ref):
    h = jnp.maximum(p_ref[0] + p_ref[1] + b2_ref[...], 0.0)
    o_ref[...] = jnp.sum(h * w3_ref[...], axis=1, keepdims=True) + b3_ref[...]


def kernel(x, adj, W1, b1, W2, b2, W3, b3):
    src = adj[0]
    dst = adj[1]
    pad = E_PAD - E
    src_p = jnp.concatenate([src, jnp.zeros((pad,), jnp.int32)])
    dst_p = jnp.concatenate([dst, jnp.full((pad,), JUNK_ROW, jnp.int32)])
    src_p = src_p.reshape(NW, NCHUNK, CHUNK)
    dst_p = dst_p.reshape(NW, NCHUNK, CHUNK)

    parts1 = _segsum(x, src_p, dst_p)

    support2 = pl.pallas_call(
        _gc_body,
        grid=(pl.cdiv(N, BM),),
        in_specs=[
            pl.BlockSpec((NC, BM, D), lambda i: (0, i, 0)),
            pl.BlockSpec((D, 2 * D), lambda i: (0, 0)),
            pl.BlockSpec((1, 2 * D), lambda i: (0, 0)),
            pl.BlockSpec((2 * D, D), lambda i: (0, 0)),
        ],
        out_specs=pl.BlockSpec((BM, D), lambda i: (i, 0)),
        out_shape=jax.ShapeDtypeStruct((N, D), jnp.float32),
    )(parts1, W1, b1.reshape(1, -1), W2)

    parts2 = _segsum(support2, src_p, dst_p)

    out = pl.pallas_call(
        _fin_body,
        grid=(pl.cdiv(N, BM),),
        in_specs=[
            pl.BlockSpec((NC, BM, D), lambda i: (0, i, 0)),
            pl.BlockSpec((1, D), lambda i: (0, 0)),
            pl.BlockSpec((1, D), lambda i: (0, 0)),
            pl.BlockSpec((1, 1), lambda i: (0, 0)),
        ],
        out_specs=pl.BlockSpec((BM, 1), lambda i: (i, 0)),
        out_shape=jax.ShapeDtypeStruct((N, 1), jnp.float32),
    )(parts2, b2.reshape(1, -1), W3.T, b3.reshape(1, 1))

    return out
